# baseline (speedup 1.0000x reference)
import functools

import jax
import jax.numpy as jnp
from jax import lax
from jax.experimental import pallas as pl
from jax.experimental.pallas import tpu as pltpu
from jax.experimental.pallas import tpu_sc as plsc

N_EMB = 8192
DIM = 256
N_BATCH = 8
N_TOK = 8192
BETA_C = 0.25

NT = 512
CH = 2048
NZ = N_TOK // NT
NM = N_EMB // CH


RS = NT // NM


def _argmin_body(z_ref, w_ref, enc_ref, vq_ref, com_ref,
                 w2_ref, col_ref, wt_ref, m_ref, accz_ref, accm_ref):
    i = pl.program_id(0)
    j = pl.program_id(1)
    js = pl.ds(j * CH, CH)
    ib = lax.rem(i, 2)
    pb = lax.rem(i + 1, 2)

    @pl.when(i == 0)
    def _():
        wblk = w_ref[js, :]
        wt_ref[:, js] = wblk.T
        w2_ref[0, js] = 0.5 * jnp.sum(wblk * wblk, axis=1)
        col_ref[0, js] = (
            lax.broadcasted_iota(jnp.int32, (1, CH), 1) + j * CH
        ).astype(jnp.float32)[0]

    @pl.when(i < NZ)
    def _():
        z = z_ref[...]
        dot = lax.dot_general(z, wt_ref[:, js], (((1,), (0,)), ((), ())),
                              precision=lax.Precision.DEFAULT,
                              preferred_element_type=jnp.float32)
        m_ref[ib, :, js] = dot - w2_ref[0, js][None, :]

        @pl.when(j == 0)
        def _():
            z2 = jnp.sum(z * z)
            prev = jnp.where(i == 0, 0.0, accz_ref[0, 0])
            accz_ref[0, 0] = prev + z2

    @pl.when(i > 0)
    def _():
        rs = pl.ds(j * RS, RS)
        mrows = m_ref[pb, rs, :]
        maxv = jnp.max(mrows, axis=1, keepdims=True)
        masked = jnp.where(m_ref[pb, rs, :] == maxv, col_ref[0, :][None, :],
                           jnp.float32(3e38))
        enc = jnp.min(masked, axis=1).astype(jnp.int32)
        enc_ref[pl.ds((i - 1) * NT + j * RS, RS)] = enc
        prev = jnp.where((i == 1) & (j == 0), 0.0, accm_ref[0, 0])
        accm_ref[0, 0] = prev + jnp.sum(maxv[:, 0])

        @pl.when((i == NZ) & (j == NM - 1))
        def _():
            vq = ((accz_ref[0, 0] - 2.0 * accm_ref[0, 0])
                  / jnp.float32(N_TOK * DIM))
            vq_ref[0, 0] = vq
            com_ref[0, 0] = BETA_C * vq


_argmin_call = pl.pallas_call(
    _argmin_body,
    grid=(NZ + 1, NM),
    in_specs=[
        pl.BlockSpec((NT, DIM), lambda i, j: (jnp.minimum(i, NZ - 1), 0)),
        pl.BlockSpec((N_EMB, DIM), lambda i, j: (0, 0)),
    ],
    out_specs=[
        pl.BlockSpec((N_TOK,), lambda i, j: (0,)),
        pl.BlockSpec(memory_space=pltpu.SMEM),
        pl.BlockSpec(memory_space=pltpu.SMEM),
    ],
    out_shape=[
        jax.ShapeDtypeStruct((N_TOK,), jnp.int32),
        jax.ShapeDtypeStruct((1, 1), jnp.float32),
        jax.ShapeDtypeStruct((1, 1), jnp.float32),
    ],
    scratch_shapes=[
        pltpu.VMEM((1, N_EMB), jnp.float32),
        pltpu.VMEM((1, N_EMB), jnp.float32),
        pltpu.VMEM((DIM, N_EMB), jnp.float32),
        pltpu.VMEM((2, NT, N_EMB), jnp.float32),
        pltpu.SMEM((1, 1), jnp.float32),
        pltpu.SMEM((1, 1), jnp.float32),
    ],
)

NC, NS = 2, 16
NW = NC * NS
TOK_PER = N_TOK // NW
HIST = N_BATCH * N_EMB
HIST_PER = HIST // NS
IDXW = 128
NIDX = TOK_PER // IDXW

def _sc_gather_hist_body(enc_hbm, bids_hbm, w_hbm, quant_hbm, counts_hbm,
                         idx_v, bid_v, flat_v, ones_v, rows_v, buf_v,
                         hist_sh, sem):
    c = lax.axis_index("c")
    s = lax.axis_index("s")
    wid = s * NC + c
    base = wid * TOK_PER

    pltpu.sync_copy(enc_hbm.at[pl.ds(wid * NIDX, NIDX)], idx_v)
    pltpu.sync_copy(bids_hbm.at[pl.ds(wid * NIDX, NIDX)], bid_v)

    gathers = [
        pltpu.async_copy(w_hbm.at[idx_v.at[k]],
                         rows_v.at[pl.ds(k * IDXW, IDXW)], sem)
        for k in range(NIDX)
    ]

    for k in range(NIDX):
        for t in range(IDXW // 16):
            sl = pl.ds(t * 16, 16)
            flat_v[k, sl] = bid_v[k, sl] * N_EMB + idx_v[k, sl]
    for t in range(IDXW // 16):
        ones_v[pl.ds(t * 16, 16)] = jnp.full((16,), 1.0, jnp.float32)

    for t in range(HIST_PER // 16):
        buf_v[pl.ds(t * 16, 16)] = jnp.zeros((16,), jnp.float32)
    pltpu.sync_copy(buf_v, hist_sh.at[pl.ds(s * HIST_PER, HIST_PER)])
    plsc.subcore_barrier()

    for k in range(NIDX):
        pltpu.sync_copy(ones_v, hist_sh.at[flat_v.at[k]], add=True)

    for g in gathers:
        g.wait()
    pltpu.sync_copy(rows_v, quant_hbm.at[pl.ds(base, TOK_PER)])
    plsc.subcore_barrier()

    pltpu.sync_copy(hist_sh.at[pl.ds(s * HIST_PER, HIST_PER)], buf_v)
    pltpu.sync_copy(buf_v, counts_hbm.at[c, pl.ds(s * HIST_PER, HIST_PER)])


@functools.lru_cache(maxsize=1)
def _sc_call():
    mesh = plsc.VectorSubcoreMesh(core_axis_name="c", subcore_axis_name="s",
                                  num_cores=NC, num_subcores=NS)
    return functools.partial(
        pl.kernel,
        out_type=[
            jax.ShapeDtypeStruct((N_TOK, DIM), jnp.float32),
            jax.ShapeDtypeStruct((NC, HIST), jnp.float32),
        ],
        mesh=mesh,
        scratch_types=[
            pltpu.VMEM((NIDX, IDXW), jnp.int32),
            pltpu.VMEM((NIDX, IDXW), jnp.int32),
            pltpu.VMEM((NIDX, IDXW), jnp.int32),
            pltpu.VMEM((IDXW,), jnp.float32),
            pltpu.VMEM((TOK_PER, DIM), jnp.float32),
            pltpu.VMEM((HIST_PER,), jnp.float32),
            pltpu.VMEM_SHARED((HIST,), jnp.float32),
            pltpu.SemaphoreType.DMA,
        ],
    )(_sc_gather_hist_body)


def _stats_body(counts_ref, perp_ref, ent_ref, uniq_ref, util_ref, buniq_ref):
    c = counts_ref[0] + counts_ref[1]
    totals = jnp.sum(c, axis=1, keepdims=True)
    probs = c / jnp.maximum(totals, 1.0)
    ent = -jnp.sum(probs * jnp.log(probs + 1e-10), axis=1)
    perp = jnp.exp(ent)
    uniq = jnp.sum((c > 0).astype(jnp.float32), axis=1)
    avg_perp = jnp.sum(perp) / N_BATCH
    avg_ent = jnp.sum(ent) / N_BATCH
    avg_uniq = jnp.sum(uniq) / N_BATCH
    perp_ref[0, 0] = avg_perp
    ent_ref[0, 0] = avg_ent
    uniq_ref[0, 0] = avg_uniq
    util_ref[0, 0] = avg_uniq / N_EMB * 100.0
    per_code = jnp.sum(c, axis=0)
    buniq_ref[0, 0] = jnp.sum((per_code > 0).astype(jnp.int32))


_stats_call = pl.pallas_call(
    _stats_body,
    out_specs=[pl.BlockSpec(memory_space=pltpu.SMEM)] * 5,
    out_shape=[jax.ShapeDtypeStruct((1, 1), jnp.float32)] * 4
    + [jax.ShapeDtypeStruct((1, 1), jnp.int32)],
)


def kernel(z_feats, batch_ids, W):
    enc, vq2d, com2d = _argmin_call(z_feats, W)
    enc2d = enc.reshape(N_TOK // IDXW, IDXW)
    bids2d = batch_ids.reshape(N_TOK // IDXW, IDXW)
    quant, counts = _sc_call()(enc2d, bids2d, W)
    perp2d, ent2d, uniq2d, util2d, buniq2d = _stats_call(
        counts.reshape(NC, N_BATCH, N_EMB))
    return (quant, vq2d[0, 0], com2d[0, 0], enc,
            perp2d[0, 0], ent2d[0, 0], uniq2d[0, 0], util2d[0, 0],
            buniq2d[0, 0])


# speedup vs baseline: 1.1383x; 1.1383x over previous
import functools

import jax
import jax.numpy as jnp
from jax import lax
from jax.experimental import pallas as pl
from jax.experimental.pallas import tpu as pltpu
from jax.experimental.pallas import tpu_sc as plsc

N_EMB = 8192
DIM = 256
N_BATCH = 8
N_TOK = 8192
BETA_C = 0.25

NT = 1024
CH = 2048
NZ = N_TOK // NT
NM = N_EMB // CH


def _argmin_body(z_ref, w_ref, enc_ref, vq_ref, com_ref,
                 w2_ref, col_ref, wt_ref, m_ref, acc_ref):
    i = pl.program_id(0)
    j = pl.program_id(1)
    js = pl.ds(j * CH, CH)
    z = z_ref[...]

    @pl.when(i == 0)
    def _():
        wblk = w_ref[js, :]
        wt_ref[:, js] = wblk.T
        w2_ref[0, js] = 0.5 * jnp.sum(wblk * wblk, axis=1)
        col_ref[0, js] = (
            lax.broadcasted_iota(jnp.int32, (1, CH), 1) + j * CH
        ).astype(jnp.float32)[0]

    dot = lax.dot_general(z, wt_ref[:, js], (((1,), (0,)), ((), ())),
                          precision=lax.Precision.DEFAULT,
                          preferred_element_type=jnp.float32)
    m_ref[:, js] = dot - w2_ref[0, js][None, :]

    @pl.when(j == NM - 1)
    def _():
        m = m_ref[...]
        maxv = jnp.max(m, axis=1, keepdims=True)
        masked = jnp.where(m_ref[...] == maxv, col_ref[0, :][None, :],
                           jnp.float32(3e38))
        enc = jnp.min(masked, axis=1).astype(jnp.int32)
        enc_ref[pl.ds(i * NT, NT)] = enc
        z2 = jnp.sum(z * z, axis=1)
        part = jnp.sum(z2) - 2.0 * jnp.sum(maxv[:, 0])
        prev = jnp.where(i == 0, 0.0, acc_ref[0, 0])
        acc_ref[0, 0] = prev + part

        @pl.when(i == NZ - 1)
        def _():
            vq = acc_ref[0, 0] / jnp.float32(N_TOK * DIM)
            vq_ref[0, 0] = vq
            com_ref[0, 0] = BETA_C * vq


_argmin_call = pl.pallas_call(
    _argmin_body,
    grid=(NZ, NM),
    in_specs=[
        pl.BlockSpec((NT, DIM), lambda i, j: (i, 0)),
        pl.BlockSpec((N_EMB, DIM), lambda i, j: (0, 0)),
    ],
    out_specs=[
        pl.BlockSpec((N_TOK,), lambda i, j: (0,)),
        pl.BlockSpec(memory_space=pltpu.SMEM),
        pl.BlockSpec(memory_space=pltpu.SMEM),
    ],
    out_shape=[
        jax.ShapeDtypeStruct((N_TOK,), jnp.int32),
        jax.ShapeDtypeStruct((1, 1), jnp.float32),
        jax.ShapeDtypeStruct((1, 1), jnp.float32),
    ],
    scratch_shapes=[
        pltpu.VMEM((1, N_EMB), jnp.float32),
        pltpu.VMEM((1, N_EMB), jnp.float32),
        pltpu.VMEM((DIM, N_EMB), jnp.float32),
        pltpu.VMEM((NT, N_EMB), jnp.float32),
        pltpu.SMEM((1, 1), jnp.float32),
    ],
)

NC, NS = 2, 16
NW = NC * NS
TOK_PER = N_TOK // NW
HIST = N_BATCH * N_EMB
HIST_PER = HIST // NS
IDXW = 128
NIDX = TOK_PER // IDXW

def _sc_gather_hist_body(enc_hbm, bids_hbm, w_hbm, quant_hbm, counts_hbm,
                         idx_v, bid_v, flat_v, ones_v, rows_v, buf_v,
                         hist_sh, sem):
    c = lax.axis_index("c")
    s = lax.axis_index("s")
    wid = s * NC + c
    base = wid * TOK_PER

    pltpu.sync_copy(enc_hbm.at[pl.ds(wid * NIDX, NIDX)], idx_v)
    pltpu.sync_copy(bids_hbm.at[pl.ds(wid * NIDX, NIDX)], bid_v)

    gathers = [
        pltpu.async_copy(w_hbm.at[idx_v.at[k]],
                         rows_v.at[pl.ds(k * IDXW, IDXW)], sem)
        for k in range(NIDX)
    ]

    for k in range(NIDX):
        for t in range(IDXW // 16):
            sl = pl.ds(t * 16, 16)
            flat_v[k, sl] = bid_v[k, sl] * N_EMB + idx_v[k, sl]
    for t in range(IDXW // 16):
        ones_v[pl.ds(t * 16, 16)] = jnp.full((16,), 1.0, jnp.float32)

    for t in range(HIST_PER // 16):
        buf_v[pl.ds(t * 16, 16)] = jnp.zeros((16,), jnp.float32)
    pltpu.sync_copy(buf_v, hist_sh.at[pl.ds(s * HIST_PER, HIST_PER)])
    plsc.subcore_barrier()

    for k in range(NIDX):
        pltpu.sync_copy(ones_v, hist_sh.at[flat_v.at[k]], add=True)

    for g in gathers:
        g.wait()
    pltpu.sync_copy(rows_v, quant_hbm.at[pl.ds(base, TOK_PER)])
    plsc.subcore_barrier()

    pltpu.sync_copy(hist_sh.at[pl.ds(s * HIST_PER, HIST_PER)], buf_v)
    pltpu.sync_copy(buf_v, counts_hbm.at[c, pl.ds(s * HIST_PER, HIST_PER)])


@functools.lru_cache(maxsize=1)
def _sc_call():
    mesh = plsc.VectorSubcoreMesh(core_axis_name="c", subcore_axis_name="s",
                                  num_cores=NC, num_subcores=NS)
    return functools.partial(
        pl.kernel,
        out_type=[
            jax.ShapeDtypeStruct((N_TOK, DIM), jnp.float32),
            jax.ShapeDtypeStruct((NC, HIST), jnp.float32),
        ],
        mesh=mesh,
        scratch_types=[
            pltpu.VMEM((NIDX, IDXW), jnp.int32),
            pltpu.VMEM((NIDX, IDXW), jnp.int32),
            pltpu.VMEM((NIDX, IDXW), jnp.int32),
            pltpu.VMEM((IDXW,), jnp.float32),
            pltpu.VMEM((TOK_PER, DIM), jnp.float32),
            pltpu.VMEM((HIST_PER,), jnp.float32),
            pltpu.VMEM_SHARED((HIST,), jnp.float32),
            pltpu.SemaphoreType.DMA,
        ],
    )(_sc_gather_hist_body)


def _stats_body(counts_ref, perp_ref, ent_ref, uniq_ref, util_ref, buniq_ref):
    c = counts_ref[0] + counts_ref[1]
    totals = jnp.sum(c, axis=1, keepdims=True)
    probs = c / jnp.maximum(totals, 1.0)
    ent = -jnp.sum(probs * jnp.log(probs + 1e-10), axis=1)
    perp = jnp.exp(ent)
    uniq = jnp.sum((c > 0).astype(jnp.float32), axis=1)
    avg_perp = jnp.sum(perp) / N_BATCH
    avg_ent = jnp.sum(ent) / N_BATCH
    avg_uniq = jnp.sum(uniq) / N_BATCH
    perp_ref[0, 0] = avg_perp
    ent_ref[0, 0] = avg_ent
    uniq_ref[0, 0] = avg_uniq
    util_ref[0, 0] = avg_uniq / N_EMB * 100.0
    per_code = jnp.sum(c, axis=0)
    buniq_ref[0, 0] = jnp.sum((per_code > 0).astype(jnp.int32))


_stats_call = pl.pallas_call(
    _stats_body,
    out_specs=[pl.BlockSpec(memory_space=pltpu.SMEM)] * 5,
    out_shape=[jax.ShapeDtypeStruct((1, 1), jnp.float32)] * 4
    + [jax.ShapeDtypeStruct((1, 1), jnp.int32)],
)


def kernel(z_feats, batch_ids, W):
    enc, vq2d, com2d = _argmin_call(z_feats, W)
    enc2d = enc.reshape(N_TOK // IDXW, IDXW)
    bids2d = batch_ids.reshape(N_TOK // IDXW, IDXW)
    quant, counts = _sc_call()(enc2d, bids2d, W)
    perp2d, ent2d, uniq2d, util2d, buniq2d = _stats_call(
        counts.reshape(NC, N_BATCH, N_EMB))
    return (quant, vq2d[0, 0], com2d[0, 0], enc,
            perp2d[0, 0], ent2d[0, 0], uniq2d[0, 0], util2d[0, 0],
            buniq2d[0, 0])


# speedup vs baseline: 1.2025x; 1.0564x over previous
import functools

import jax
import jax.numpy as jnp
from jax import lax
from jax.experimental import pallas as pl
from jax.experimental.pallas import tpu as pltpu
from jax.experimental.pallas import tpu_sc as plsc

N_EMB = 8192
DIM = 256
N_BATCH = 8
N_TOK = 8192
BETA_C = 0.25

NT = 1024
CH = 4096
NZ = N_TOK // NT
NM = N_EMB // CH


def _argmin_body(z_ref, w_ref, enc_ref, vq_ref, com_ref,
                 w2_ref, col_ref, wt_ref, m_ref, acc_ref):
    i = pl.program_id(0)
    j = pl.program_id(1)
    js = pl.ds(j * CH, CH)
    z = z_ref[...]

    @pl.when(i == 0)
    def _():
        wblk = w_ref[js, :]
        wt_ref[:, js] = wblk.T
        w2_ref[0, js] = 0.5 * jnp.sum(wblk * wblk, axis=1)
        col_ref[0, js] = (
            lax.broadcasted_iota(jnp.int32, (1, CH), 1) + j * CH
        ).astype(jnp.float32)[0]

    dot = lax.dot_general(z, wt_ref[:, js], (((1,), (0,)), ((), ())),
                          precision=lax.Precision.DEFAULT,
                          preferred_element_type=jnp.float32)
    m_ref[:, js] = dot - w2_ref[0, js][None, :]

    @pl.when(j == NM - 1)
    def _():
        m = m_ref[...]
        maxv = jnp.max(m, axis=1, keepdims=True)
        masked = jnp.where(m_ref[...] == maxv, col_ref[0, :][None, :],
                           jnp.float32(3e38))
        enc = jnp.min(masked, axis=1).astype(jnp.int32)
        enc_ref[pl.ds(i * NT, NT)] = enc
        z2 = jnp.sum(z * z, axis=1)
        part = jnp.sum(z2) - 2.0 * jnp.sum(maxv[:, 0])
        prev = jnp.where(i == 0, 0.0, acc_ref[0, 0])
        acc_ref[0, 0] = prev + part

        @pl.when(i == NZ - 1)
        def _():
            vq = acc_ref[0, 0] / jnp.float32(N_TOK * DIM)
            vq_ref[0, 0] = vq
            com_ref[0, 0] = BETA_C * vq


_argmin_call = pl.pallas_call(
    _argmin_body,
    grid=(NZ, NM),
    in_specs=[
        pl.BlockSpec((NT, DIM), lambda i, j: (i, 0)),
        pl.BlockSpec((N_EMB, DIM), lambda i, j: (0, 0)),
    ],
    out_specs=[
        pl.BlockSpec((N_TOK,), lambda i, j: (0,)),
        pl.BlockSpec(memory_space=pltpu.SMEM),
        pl.BlockSpec(memory_space=pltpu.SMEM),
    ],
    out_shape=[
        jax.ShapeDtypeStruct((N_TOK,), jnp.int32),
        jax.ShapeDtypeStruct((1, 1), jnp.float32),
        jax.ShapeDtypeStruct((1, 1), jnp.float32),
    ],
    scratch_shapes=[
        pltpu.VMEM((1, N_EMB), jnp.float32),
        pltpu.VMEM((1, N_EMB), jnp.float32),
        pltpu.VMEM((DIM, N_EMB), jnp.float32),
        pltpu.VMEM((NT, N_EMB), jnp.float32),
        pltpu.SMEM((1, 1), jnp.float32),
    ],
)

NC, NS = 2, 16
NW = NC * NS
TOK_PER = N_TOK // NW
HIST = N_BATCH * N_EMB
HIST_PER = HIST // NS
IDXW = 128
NIDX = TOK_PER // IDXW

def _sc_gather_hist_body(enc_hbm, bids_hbm, w_hbm, quant_hbm, counts_hbm,
                         idx_v, bid_v, flat_v, ones_v, rows_v, buf_v,
                         hist_sh, sem):
    c = lax.axis_index("c")
    s = lax.axis_index("s")
    wid = s * NC + c
    base = wid * TOK_PER

    pltpu.sync_copy(enc_hbm.at[pl.ds(wid * NIDX, NIDX)], idx_v)
    pltpu.sync_copy(bids_hbm.at[pl.ds(wid * NIDX, NIDX)], bid_v)

    gathers = [
        pltpu.async_copy(w_hbm.at[idx_v.at[k]],
                         rows_v.at[pl.ds(k * IDXW, IDXW)], sem)
        for k in range(NIDX)
    ]

    for k in range(NIDX):
        for t in range(IDXW // 16):
            sl = pl.ds(t * 16, 16)
            flat_v[k, sl] = bid_v[k, sl] * N_EMB + idx_v[k, sl]
    for t in range(IDXW // 16):
        ones_v[pl.ds(t * 16, 16)] = jnp.full((16,), 1.0, jnp.float32)

    for t in range(HIST_PER // 16):
        buf_v[pl.ds(t * 16, 16)] = jnp.zeros((16,), jnp.float32)
    pltpu.sync_copy(buf_v, hist_sh.at[pl.ds(s * HIST_PER, HIST_PER)])
    plsc.subcore_barrier()

    for k in range(NIDX):
        pltpu.sync_copy(ones_v, hist_sh.at[flat_v.at[k]], add=True)

    for g in gathers:
        g.wait()
    pltpu.sync_copy(rows_v, quant_hbm.at[pl.ds(base, TOK_PER)])
    plsc.subcore_barrier()

    pltpu.sync_copy(hist_sh.at[pl.ds(s * HIST_PER, HIST_PER)], buf_v)
    pltpu.sync_copy(buf_v, counts_hbm.at[c, pl.ds(s * HIST_PER, HIST_PER)])


@functools.lru_cache(maxsize=1)
def _sc_call():
    mesh = plsc.VectorSubcoreMesh(core_axis_name="c", subcore_axis_name="s",
                                  num_cores=NC, num_subcores=NS)
    return functools.partial(
        pl.kernel,
        out_type=[
            jax.ShapeDtypeStruct((N_TOK, DIM), jnp.float32),
            jax.ShapeDtypeStruct((NC, HIST), jnp.float32),
        ],
        mesh=mesh,
        scratch_types=[
            pltpu.VMEM((NIDX, IDXW), jnp.int32),
            pltpu.VMEM((NIDX, IDXW), jnp.int32),
            pltpu.VMEM((NIDX, IDXW), jnp.int32),
            pltpu.VMEM((IDXW,), jnp.float32),
            pltpu.VMEM((TOK_PER, DIM), jnp.float32),
            pltpu.VMEM((HIST_PER,), jnp.float32),
            pltpu.VMEM_SHARED((HIST,), jnp.float32),
            pltpu.SemaphoreType.DMA,
        ],
    )(_sc_gather_hist_body)


def _stats_body(counts_ref, perp_ref, ent_ref, uniq_ref, util_ref, buniq_ref):
    c = counts_ref[0] + counts_ref[1]
    totals = jnp.sum(c, axis=1, keepdims=True)
    probs = c / jnp.maximum(totals, 1.0)
    ent = -jnp.sum(probs * jnp.log(probs + 1e-10), axis=1)
    perp = jnp.exp(ent)
    uniq = jnp.sum((c > 0).astype(jnp.float32), axis=1)
    avg_perp = jnp.sum(perp) / N_BATCH
    avg_ent = jnp.sum(ent) / N_BATCH
    avg_uniq = jnp.sum(uniq) / N_BATCH
    perp_ref[0, 0] = avg_perp
    ent_ref[0, 0] = avg_ent
    uniq_ref[0, 0] = avg_uniq
    util_ref[0, 0] = avg_uniq / N_EMB * 100.0
    per_code = jnp.sum(c, axis=0)
    buniq_ref[0, 0] = jnp.sum((per_code > 0).astype(jnp.int32))


_stats_call = pl.pallas_call(
    _stats_body,
    out_specs=[pl.BlockSpec(memory_space=pltpu.SMEM)] * 5,
    out_shape=[jax.ShapeDtypeStruct((1, 1), jnp.float32)] * 4
    + [jax.ShapeDtypeStruct((1, 1), jnp.int32)],
)


def kernel(z_feats, batch_ids, W):
    enc, vq2d, com2d = _argmin_call(z_feats, W)
    enc2d = enc.reshape(N_TOK // IDXW, IDXW)
    bids2d = batch_ids.reshape(N_TOK // IDXW, IDXW)
    quant, counts = _sc_call()(enc2d, bids2d, W)
    perp2d, ent2d, uniq2d, util2d, buniq2d = _stats_call(
        counts.reshape(NC, N_BATCH, N_EMB))
    return (quant, vq2d[0, 0], com2d[0, 0], enc,
            perp2d[0, 0], ent2d[0, 0], uniq2d[0, 0], util2d[0, 0],
            buniq2d[0, 0])


# speedup vs baseline: 1.2311x; 1.0238x over previous
import functools

import jax
import jax.numpy as jnp
from jax import lax
from jax.experimental import pallas as pl
from jax.experimental.pallas import tpu as pltpu
from jax.experimental.pallas import tpu_sc as plsc

N_EMB = 8192
DIM = 256
N_BATCH = 8
N_TOK = 8192
BETA_C = 0.25

NT = 1024
CH = 8192
NZ = N_TOK // NT
NM = N_EMB // CH


def _argmin_body(z_ref, w_ref, enc_ref, vq_ref, com_ref,
                 w2_ref, col_ref, wt_ref, m_ref, acc_ref):
    i = pl.program_id(0)
    j = pl.program_id(1)
    js = pl.ds(j * CH, CH)
    z = z_ref[...]

    @pl.when(i == 0)
    def _():
        wblk = w_ref[js, :]
        wt_ref[:, js] = wblk.T
        w2_ref[0, js] = 0.5 * jnp.sum(wblk * wblk, axis=1)
        col_ref[0, js] = (
            lax.broadcasted_iota(jnp.int32, (1, CH), 1) + j * CH
        ).astype(jnp.float32)[0]

    dot = lax.dot_general(z, wt_ref[:, js], (((1,), (0,)), ((), ())),
                          precision=lax.Precision.DEFAULT,
                          preferred_element_type=jnp.float32)
    m_ref[:, js] = dot - w2_ref[0, js][None, :]

    @pl.when(j == NM - 1)
    def _():
        m = m_ref[...]
        maxv = jnp.max(m, axis=1, keepdims=True)
        masked = jnp.where(m_ref[...] == maxv, col_ref[0, :][None, :],
                           jnp.float32(3e38))
        enc = jnp.min(masked, axis=1).astype(jnp.int32)
        enc_ref[pl.ds(i * NT, NT)] = enc
        z2 = jnp.sum(z * z, axis=1)
        part = jnp.sum(z2) - 2.0 * jnp.sum(maxv[:, 0])
        prev = jnp.where(i == 0, 0.0, acc_ref[0, 0])
        acc_ref[0, 0] = prev + part

        @pl.when(i == NZ - 1)
        def _():
            vq = acc_ref[0, 0] / jnp.float32(N_TOK * DIM)
            vq_ref[0, 0] = vq
            com_ref[0, 0] = BETA_C * vq


_argmin_call = pl.pallas_call(
    _argmin_body,
    grid=(NZ, NM),
    in_specs=[
        pl.BlockSpec((NT, DIM), lambda i, j: (i, 0)),
        pl.BlockSpec((N_EMB, DIM), lambda i, j: (0, 0)),
    ],
    out_specs=[
        pl.BlockSpec((N_TOK,), lambda i, j: (0,)),
        pl.BlockSpec(memory_space=pltpu.SMEM),
        pl.BlockSpec(memory_space=pltpu.SMEM),
    ],
    out_shape=[
        jax.ShapeDtypeStruct((N_TOK,), jnp.int32),
        jax.ShapeDtypeStruct((1, 1), jnp.float32),
        jax.ShapeDtypeStruct((1, 1), jnp.float32),
    ],
    scratch_shapes=[
        pltpu.VMEM((1, N_EMB), jnp.float32),
        pltpu.VMEM((1, N_EMB), jnp.float32),
        pltpu.VMEM((DIM, N_EMB), jnp.float32),
        pltpu.VMEM((NT, N_EMB), jnp.float32),
        pltpu.SMEM((1, 1), jnp.float32),
    ],
)

NC, NS = 2, 16
NW = NC * NS
TOK_PER = N_TOK // NW
HIST = N_BATCH * N_EMB
HIST_PER = HIST // NS
IDXW = 128
NIDX = TOK_PER // IDXW

def _sc_gather_hist_body(enc_hbm, bids_hbm, w_hbm, quant_hbm, counts_hbm,
                         idx_v, bid_v, flat_v, ones_v, rows_v, buf_v,
                         hist_sh, sem):
    c = lax.axis_index("c")
    s = lax.axis_index("s")
    wid = s * NC + c
    base = wid * TOK_PER

    pltpu.sync_copy(enc_hbm.at[pl.ds(wid * NIDX, NIDX)], idx_v)
    pltpu.sync_copy(bids_hbm.at[pl.ds(wid * NIDX, NIDX)], bid_v)

    gathers = [
        pltpu.async_copy(w_hbm.at[idx_v.at[k]],
                         rows_v.at[pl.ds(k * IDXW, IDXW)], sem)
        for k in range(NIDX)
    ]

    for k in range(NIDX):
        for t in range(IDXW // 16):
            sl = pl.ds(t * 16, 16)
            flat_v[k, sl] = bid_v[k, sl] * N_EMB + idx_v[k, sl]
    for t in range(IDXW // 16):
        ones_v[pl.ds(t * 16, 16)] = jnp.full((16,), 1.0, jnp.float32)

    for t in range(HIST_PER // 16):
        buf_v[pl.ds(t * 16, 16)] = jnp.zeros((16,), jnp.float32)
    pltpu.sync_copy(buf_v, hist_sh.at[pl.ds(s * HIST_PER, HIST_PER)])
    plsc.subcore_barrier()

    for k in range(NIDX):
        pltpu.sync_copy(ones_v, hist_sh.at[flat_v.at[k]], add=True)

    for g in gathers:
        g.wait()
    pltpu.sync_copy(rows_v, quant_hbm.at[pl.ds(base, TOK_PER)])
    plsc.subcore_barrier()

    pltpu.sync_copy(hist_sh.at[pl.ds(s * HIST_PER, HIST_PER)], buf_v)
    pltpu.sync_copy(buf_v, counts_hbm.at[c, pl.ds(s * HIST_PER, HIST_PER)])


@functools.lru_cache(maxsize=1)
def _sc_call():
    mesh = plsc.VectorSubcoreMesh(core_axis_name="c", subcore_axis_name="s",
                                  num_cores=NC, num_subcores=NS)
    return functools.partial(
        pl.kernel,
        out_type=[
            jax.ShapeDtypeStruct((N_TOK, DIM), jnp.float32),
            jax.ShapeDtypeStruct((NC, HIST), jnp.float32),
        ],
        mesh=mesh,
        scratch_types=[
            pltpu.VMEM((NIDX, IDXW), jnp.int32),
            pltpu.VMEM((NIDX, IDXW), jnp.int32),
            pltpu.VMEM((NIDX, IDXW), jnp.int32),
            pltpu.VMEM((IDXW,), jnp.float32),
            pltpu.VMEM((TOK_PER, DIM), jnp.float32),
            pltpu.VMEM((HIST_PER,), jnp.float32),
            pltpu.VMEM_SHARED((HIST,), jnp.float32),
            pltpu.SemaphoreType.DMA,
        ],
    )(_sc_gather_hist_body)


def _stats_body(counts_ref, perp_ref, ent_ref, uniq_ref, util_ref, buniq_ref):
    c = counts_ref[0] + counts_ref[1]
    totals = jnp.sum(c, axis=1, keepdims=True)
    probs = c / jnp.maximum(totals, 1.0)
    ent = -jnp.sum(probs * jnp.log(probs + 1e-10), axis=1)
    perp = jnp.exp(ent)
    uniq = jnp.sum((c > 0).astype(jnp.float32), axis=1)
    avg_perp = jnp.sum(perp) / N_BATCH
    avg_ent = jnp.sum(ent) / N_BATCH
    avg_uniq = jnp.sum(uniq) / N_BATCH
    perp_ref[0, 0] = avg_perp
    ent_ref[0, 0] = avg_ent
    uniq_ref[0, 0] = avg_uniq
    util_ref[0, 0] = avg_uniq / N_EMB * 100.0
    per_code = jnp.sum(c, axis=0)
    buniq_ref[0, 0] = jnp.sum((per_code > 0).astype(jnp.int32))


_stats_call = pl.pallas_call(
    _stats_body,
    out_specs=[pl.BlockSpec(memory_space=pltpu.SMEM)] * 5,
    out_shape=[jax.ShapeDtypeStruct((1, 1), jnp.float32)] * 4
    + [jax.ShapeDtypeStruct((1, 1), jnp.int32)],
)


def kernel(z_feats, batch_ids, W):
    enc, vq2d, com2d = _argmin_call(z_feats, W)
    enc2d = enc.reshape(N_TOK // IDXW, IDXW)
    bids2d = batch_ids.reshape(N_TOK // IDXW, IDXW)
    quant, counts = _sc_call()(enc2d, bids2d, W)
    perp2d, ent2d, uniq2d, util2d, buniq2d = _stats_call(
        counts.reshape(NC, N_BATCH, N_EMB))
    return (quant, vq2d[0, 0], com2d[0, 0], enc,
            perp2d[0, 0], ent2d[0, 0], uniq2d[0, 0], util2d[0, 0],
            buniq2d[0, 0])
